# Initial kernel scaffold; baseline (speedup 1.0000x reference)
import jax
import jax.numpy as jnp
from jax.experimental import pallas as pl


def kernel(x, edge_index, node_emb, pos_w, pos_b, W1, att_src1, att_dst1, b1, W2, att_src2, att_dst2, b2, pool_w, pool_b, W_ih, W_hh, b_ih, b_hh, fc_w, fc_b):
    raise NotImplementedError("write your pallas kernel here")


# speedup vs baseline: 12.7340x; 12.7340x over previous
import jax
import jax.numpy as jnp
from jax.experimental import pallas as pl

_B = 1024
_T = 32
_N = 8
_FP = 8


def _dot(a, b, dims):
    return jax.lax.dot_general(a, b, (dims, ((), ())),
                               preferred_element_type=jnp.float32)


def _lrelu(x):
    return jnp.where(x >= 0, x, 0.2 * x)


def _elu(x):
    return jnp.where(x > 0, x, jnp.exp(jnp.minimum(x, 0.0)) - 1.0)


def _gat_pool_kernel(xp_ref, ei_ref, npos_ref, nb_ref, poswT_ref, posb_ref,
                     W1_ref, as1_ref, ad1_ref, b1_ref, W2_ref, as2_ref,
                     ad2_ref, b2_ref, poolw_ref, poolb_ref, out_ref):
    f32 = jnp.float32
    src = ei_ref[0:1, :]
    dst = ei_ref[1:2, :]
    iota8 = jax.lax.broadcasted_iota(jnp.int32, (_N, 64), 0)
    s_oh = (iota8 == src).astype(f32)
    d_oh = (iota8 == dst).astype(f32)
    CT = _dot(s_oh, d_oh, ((1,), (1,)))

    base = nb_ref[...] + npos_ref[...] * poswT_ref[...] + posb_ref[...]
    HB = _dot(W1_ref[...], base, ((0,), (1,)))

    h1 = []
    a1s = []
    a1d = []
    for n in range(_N):
        hn = _dot(W1_ref[...], xp_ref[n], ((0,), (0,))) + HB[:, n:n + 1]
        h1.append(hn)
        a1s.append([jnp.sum(hn[32 * h:32 * h + 32, :] * as1_ref[:, h:h + 1],
                            axis=0, keepdims=True) for h in range(2)])
        a1d.append([jnp.sum(hn[32 * h:32 * h + 32, :] * ad1_ref[:, h:h + 1],
                            axis=0, keepdims=True) for h in range(2)])

    h1p = []
    for d in range(_N):
        cd = CT[:, d:d + 1]
        mask = cd > 0
        accs = []
        for h in range(2):
            E = _lrelu(jnp.concatenate([a1s[s][h] for s in range(_N)], axis=0)
                       + a1d[d][h])
            Em = jnp.where(mask, E, -1e30)
            emax = jnp.max(Em, axis=0, keepdims=True)
            ee = jnp.where(mask, jnp.exp(E - emax) * cd, 0.0)
            den = jnp.sum(ee, axis=0, keepdims=True)
            wgt = ee / (den + 1e-16)
            acc = wgt[0:1, :] * h1[0][32 * h:32 * h + 32, :]
            for s in range(1, _N):
                acc = acc + wgt[s:s + 1, :] * h1[s][32 * h:32 * h + 32, :]
            accs.append(acc)
        h1p.append(_elu(jnp.concatenate(accs, axis=0) + b1_ref[...]))

    h2 = []
    a2s = []
    a2d = []
    for n in range(_N):
        hn = _dot(W2_ref[...], h1p[n], ((0,), (0,)))
        h2.append(hn)
        a2s.append(jnp.sum(hn * as2_ref[...], axis=0, keepdims=True))
        a2d.append(jnp.sum(hn * ad2_ref[...], axis=0, keepdims=True))

    hh = []
    for d in range(_N):
        cd = CT[:, d:d + 1]
        mask = cd > 0
        E = _lrelu(jnp.concatenate(a2s, axis=0) + a2d[d])
        Em = jnp.where(mask, E, -1e30)
        emax = jnp.max(Em, axis=0, keepdims=True)
        ee = jnp.where(mask, jnp.exp(E - emax) * cd, 0.0)
        den = jnp.sum(ee, axis=0, keepdims=True)
        wgt = ee / (den + 1e-16)
        acc = wgt[0:1, :] * h2[0]
        for s in range(1, _N):
            acc = acc + wgt[s:s + 1, :] * h2[s]
        hh.append(_elu(acc + b2_ref[...]))

    scores = jnp.concatenate(
        [jnp.sum(hh[d] * poolw_ref[...], axis=0, keepdims=True)
         for d in range(_N)], axis=0) + poolb_ref[...]
    m = jnp.max(scores, axis=0, keepdims=True)
    p = jnp.exp(scores - m)
    w = p / jnp.sum(p, axis=0, keepdims=True)
    pooled = w[0:1, :] * hh[0]
    for d in range(1, _N):
        pooled = pooled + w[d:d + 1, :] * hh[d]
    out_ref[0] = pooled


def _gru_fc_kernel(pooled_ref, Wih_ref, Whh_ref, bih_ref, bhh_ref,
                   fcw_ref, fcb_ref, out_ref):
    H = 128

    def sigmoid(x):
        return 1.0 / (1.0 + jnp.exp(-x))

    def step(t, hcur):
        xt = pooled_ref[t]
        gi = _dot(Wih_ref[...], xt, ((1,), (0,))) + bih_ref[...]
        gh = _dot(Whh_ref[...], hcur, ((1,), (0,))) + bhh_ref[...]
        r = sigmoid(gi[0:H] + gh[0:H])
        z = sigmoid(gi[H:2 * H] + gh[H:2 * H])
        n = jnp.tanh(gi[2 * H:3 * H] + r * gh[2 * H:3 * H])
        return (1.0 - z) * n + z * hcur

    hT = jax.lax.fori_loop(0, _T, step, jnp.zeros((H, _B), jnp.float32))
    out_ref[...] = _dot(fcw_ref[...], hT, ((1,), (0,))) + fcb_ref[...]


def kernel(x, edge_index, node_emb, pos_w, pos_b, W1, att_src1, att_dst1, b1,
           W2, att_src2, att_dst2, b2, pool_w, pool_b,
           W_ih, W_hh, b_ih, b_hh, fc_w, fc_b):
    f32 = jnp.float32
    B, T, N, F = x.shape

    xp = jnp.pad(x, ((0, 0), (0, 0), (0, 0), (0, _FP - F)))
    xp = xp.transpose(2, 3, 1, 0).reshape(N, _FP, T * B)
    ei = jnp.pad(edge_index.astype(jnp.int32), ((0, 6), (0, 0)))
    nb = jnp.pad(node_emb, ((0, 0), (0, _FP - F)))
    poswT = jnp.pad(pos_w, ((0, _FP - F), (0, 0))).T
    posb = jnp.pad(pos_b, (0, _FP - F)).reshape(1, _FP)
    W1p = jnp.pad(W1, ((0, _FP - F), (0, 0)))
    as1T = att_src1.T
    ad1T = att_dst1.T
    b1c = b1.reshape(64, 1)
    as2T = att_src2.T
    ad2T = att_dst2.T
    b2c = b2.reshape(32, 1)
    poolwT = pool_w.T
    poolbc = pool_b.reshape(1, 1)
    npos = jnp.array([0., 1., 2., 2., 1., 2., 1., 2.],
                     dtype=f32).reshape(8, 1)

    full = lambda shp: pl.BlockSpec(shp, lambda *_: (0,) * len(shp))
    pooled = pl.pallas_call(
        _gat_pool_kernel,
        grid=(T,),
        in_specs=[
            pl.BlockSpec((N, _FP, B), lambda i: (0, 0, i)),
            full((8, 64)), full((8, 1)), full((8, 8)), full((1, 8)),
            full((1, 8)),
            full((8, 64)), full((32, 2)), full((32, 2)), full((64, 1)),
            full((64, 32)), full((32, 1)), full((32, 1)), full((32, 1)),
            full((32, 1)), full((1, 1)),
        ],
        out_specs=pl.BlockSpec((1, 32, B), lambda i: (i, 0, 0)),
        out_shape=jax.ShapeDtypeStruct((T, 32, B), f32),
    )(xp, ei, npos, nb, poswT, posb, W1p, as1T, ad1T, b1c, W2, as2T, ad2T,
      b2c, poolwT, poolbc)

    out = pl.pallas_call(
        _gru_fc_kernel,
        in_specs=[
            full((T, 32, B)), full((384, 32)), full((384, 128)),
            full((384, 1)), full((384, 1)), full((8, 128)), full((8, 1)),
        ],
        out_specs=full((8, B)),
        out_shape=jax.ShapeDtypeStruct((8, B), f32),
    )(pooled, W_ih, W_hh, b_ih.reshape(384, 1), b_hh.reshape(384, 1),
      fc_w, fc_b.reshape(8, 1))

    return out.T
